# baseline (speedup 1.0000x reference)
import functools

import jax
import jax.numpy as jnp
from jax import lax
from jax.experimental import pallas as pl
from jax.experimental.pallas import tpu as pltpu
from jax.experimental.pallas import tpu_sc as plsc

NSUB = 16
NCORE = 2
CH = 128


def _sc_segment_sum(xcat, gsrc2, sidx2, *, n_nodes, nch_per_sub, acc_rows):
    n2 = 2 * n_nodes
    nchunk = nch_per_sub * NSUB
    zrep = acc_rows // (NSUB * CH)
    orows = n2 // NSUB
    mesh = plsc.VectorSubcoreMesh(core_axis_name="c", subcore_axis_name="s")

    @functools.partial(
        pl.kernel,
        mesh=mesh,
        out_type=jax.ShapeDtypeStruct((2 * n2, 64), jnp.float32),
        scratch_types=(
            [pltpu.VMEM((1, CH), jnp.int32)] * 2 +
            [pltpu.VMEM((CH, 64), jnp.float32)] +
            [pltpu.VMEM_SHARED((acc_rows, 64), jnp.float32)] +
            [pltpu.SemaphoreType.DMA]
        ),
        compiler_params=pltpu.CompilerParams(use_tc_tiling_on_sc=False),
    )
    def k(xcat_h, gsrc_h, sidx_h, out_h, gia, sia, b0, acc, semg):
        c = lax.axis_index("c")
        s = lax.axis_index("s")
        T = nch_per_sub

        def zv(i, carry):
            b0[i // 4, pl.ds((i % 4) * 16, 16)] = jnp.zeros(
                (16,), jnp.float32)
            return carry
        lax.fori_loop(0, CH * 4, zv, 0)

        def za(i, carry):
            pltpu.sync_copy(b0, acc.at[pl.ds((s * zrep + i) * CH, CH)])
            return carry
        lax.fori_loop(0, zrep, za, 0)
        plsc.subcore_barrier()

        def step(j, carry):
            ch = s * T + j
            pltpu.sync_copy(gsrc_h.at[pl.ds(c * nchunk + ch, 1)], gia)
            pltpu.sync_copy(sidx_h.at[pl.ds(ch, 1)], sia)
            pltpu.async_copy(xcat_h.at[gia.at[0]], b0, semg).wait()
            pltpu.sync_copy(b0, acc.at[sia.at[0]], add=True)
            return carry
        lax.fori_loop(0, T, step, 0)
        plsc.subcore_barrier()

        pltpu.sync_copy(acc.at[pl.ds(s * orows, orows)],
                        out_h.at[pl.ds(c * n2 + s * orows, orows)])

    return k(xcat, gsrc2, sidx2)


def _tc_layer(sp, xin, W, lw, b, *, relu, split_out, blk=1000):
    n = xin.shape[1]
    grid = (n // blk,)

    def body(sp_ref, x_ref, w_ref, lw_ref, b_ref, o_ref):
        w = w_ref[...]
        lw_ = lw_ref[...]
        acc = jnp.dot(x_ref[0], lw_[:64], preferred_element_type=jnp.float32)
        acc += jnp.dot(x_ref[1], lw_[64:], preferred_element_type=jnp.float32)
        for c in range(2):
            for r in range(2):
                acc += jnp.dot(sp_ref[c, r], w[r, c * 64:(c + 1) * 64],
                               preferred_element_type=jnp.float32)
        acc += b_ref[...]
        if relu:
            acc = jnp.maximum(acc, 0.0)
        if split_out:
            o_ref[0] = acc[:, :64]
            o_ref[1] = acc[:, 64:]
        else:
            o_ref[...] = acc

    if split_out:
        out_shape = jax.ShapeDtypeStruct((2, n, 64), jnp.float32)
        out_spec = pl.BlockSpec((2, blk, 64), lambda i: (0, i, 0))
    else:
        out_shape = jax.ShapeDtypeStruct((n, 128), jnp.float32)
        out_spec = pl.BlockSpec((blk, 128), lambda i: (i, 0))

    return pl.pallas_call(
        body,
        grid=grid,
        in_specs=[
            pl.BlockSpec((2, 2, blk, 64), lambda i: (0, 0, i, 0)),
            pl.BlockSpec((2, blk, 64), lambda i: (0, i, 0)),
            pl.BlockSpec((2, 128, 128), lambda i: (0, 0, 0)),
            pl.BlockSpec((128, 128), lambda i: (0, 0)),
            pl.BlockSpec((1, 128), lambda i: (0, 0)),
        ],
        out_specs=out_spec,
        out_shape=out_shape,
    )(sp, xin, W, lw, b)


def kernel(feat, edge_index, etype, W1, loopW1, b1, W2, loopW2, b2):
    n = feat.shape[0]
    e = edge_index.shape[1]
    n2 = 2 * n

    nch_per_sub = -(-e // (NSUB * CH))
    e_pad = nch_per_sub * NSUB * CH
    nchunk = e_pad // CH
    acc_rows = -(-(n2 + 1) // (NSUB * CH)) * (NSUB * CH)

    src = edge_index[0].astype(jnp.int32)
    dst = edge_index[1].astype(jnp.int32)
    et = etype.astype(jnp.int32)
    pad = e_pad - e
    gidx = jnp.concatenate([src, jnp.zeros((pad,), jnp.int32)])
    gsrc2 = jnp.concatenate([gidx, gidx + n]).reshape(2 * nchunk, CH)
    sidx2 = jnp.concatenate(
        [et * n + dst, jnp.full((pad,), n2, jnp.int32)]).reshape(nchunk, CH)

    xcat = feat.reshape(n, 2, 64).transpose(1, 0, 2)

    sc = functools.partial(_sc_segment_sum, n_nodes=n,
                           nch_per_sub=nch_per_sub, acc_rows=acc_rows)

    sp1 = sc(xcat.reshape(n2, 64), gsrc2, sidx2).reshape(2, 2, n, 64)
    h = _tc_layer(sp1, xcat, W1, loopW1, b1.reshape(1, 128),
                  relu=True, split_out=True)
    sp2 = sc(h.reshape(n2, 64), gsrc2, sidx2).reshape(2, 2, n, 64)
    out = _tc_layer(sp2, h, W2, loopW2, b2.reshape(1, 128),
                    relu=False, split_out=False)
    return out


# speedup vs baseline: 1.0428x; 1.0428x over previous
import functools

import jax
import jax.numpy as jnp
from jax import lax
from jax.experimental import pallas as pl
from jax.experimental.pallas import tpu as pltpu
from jax.experimental.pallas import tpu_sc as plsc

NSUB = 16
NCORE = 2
CH = 128


def _sc_segment_sum(xcat, gsrc2, sidx2, *, n_nodes, nch_per_sub, acc_rows):
    n2 = 2 * n_nodes
    nchunk = nch_per_sub * NSUB
    zrep = acc_rows // (NSUB * CH)
    orows = n2 // NSUB
    mesh = plsc.VectorSubcoreMesh(core_axis_name="c", subcore_axis_name="s")

    @functools.partial(
        pl.kernel,
        mesh=mesh,
        out_type=jax.ShapeDtypeStruct((2 * n2, 64), jnp.float32),
        scratch_types=(
            [pltpu.VMEM((1, CH), jnp.int32)] * 2 +
            [pltpu.VMEM((CH, 64), jnp.float32)] +
            [pltpu.VMEM_SHARED((acc_rows, 64), jnp.float32)] +
            [pltpu.SemaphoreType.DMA]
        ),
        compiler_params=pltpu.CompilerParams(use_tc_tiling_on_sc=False),
    )
    def k(xcat_h, gsrc_h, sidx_h, out_h, gia, sia, b0, acc, semg):
        c = lax.axis_index("c")
        s = lax.axis_index("s")
        T = nch_per_sub

        def zv(i, carry):
            b0[i // 4, pl.ds((i % 4) * 16, 16)] = jnp.zeros(
                (16,), jnp.float32)
            return carry
        lax.fori_loop(0, CH * 4, zv, 0)

        def za(i, carry):
            pltpu.sync_copy(b0, acc.at[pl.ds((s * zrep + i) * CH, CH)])
            return carry
        lax.fori_loop(0, zrep, za, 0)
        plsc.subcore_barrier()

        def step(j, carry):
            ch = s * T + j
            pltpu.sync_copy(gsrc_h.at[pl.ds(c * nchunk + ch, 1)], gia)
            pltpu.sync_copy(sidx_h.at[pl.ds(ch, 1)], sia)
            pltpu.async_copy(xcat_h.at[gia.at[0]], b0, semg).wait()
            pltpu.sync_copy(b0, acc.at[sia.at[0]], add=True)
            return carry
        lax.fori_loop(0, T, step, 0)
        plsc.subcore_barrier()

        pltpu.sync_copy(acc.at[pl.ds(s * orows, orows)],
                        out_h.at[pl.ds(c * n2 + s * orows, orows)])

    return k(xcat, gsrc2, sidx2)


def _tc_layer(sp, xin, W, lw, b, *, relu, blk=1000):
    n = xin.shape[0]
    grid = (n // blk,)

    def body(sp_ref, x_ref, w_ref, lw_ref, b_ref, o_ref):
        w = w_ref[...]
        acc = jnp.dot(x_ref[...], lw_ref[...],
                      preferred_element_type=jnp.float32)
        for c in range(2):
            for r in range(2):
                acc += jnp.dot(sp_ref[c, r], w[r, c * 64:(c + 1) * 64],
                               preferred_element_type=jnp.float32)
        acc += b_ref[...]
        if relu:
            acc = jnp.maximum(acc, 0.0)
        o_ref[...] = acc

    return pl.pallas_call(
        body,
        grid=grid,
        in_specs=[
            pl.BlockSpec((2, 2, blk, 64), lambda i: (0, 0, i, 0)),
            pl.BlockSpec((blk, 128), lambda i: (i, 0)),
            pl.BlockSpec((2, 128, 128), lambda i: (0, 0, 0)),
            pl.BlockSpec((128, 128), lambda i: (0, 0)),
            pl.BlockSpec((1, 128), lambda i: (0, 0)),
        ],
        out_specs=pl.BlockSpec((blk, 128), lambda i: (i, 0)),
        out_shape=jax.ShapeDtypeStruct((n, 128), jnp.float32),
    )(sp, xin, W, lw, b)


def kernel(feat, edge_index, etype, W1, loopW1, b1, W2, loopW2, b2):
    n = feat.shape[0]
    e = edge_index.shape[1]
    n2 = 2 * n

    nch_per_sub = -(-e // (NSUB * CH))
    e_pad = nch_per_sub * NSUB * CH
    nchunk = e_pad // CH
    acc_rows = -(-(n2 + 1) // (NSUB * CH)) * (NSUB * CH)

    src = edge_index[0].astype(jnp.int32)
    dst = edge_index[1].astype(jnp.int32)
    et = etype.astype(jnp.int32)
    pad = e_pad - e
    gidx = jnp.concatenate([2 * src, jnp.zeros((pad,), jnp.int32)])
    gsrc2 = jnp.concatenate([gidx, gidx + 1]).reshape(2 * nchunk, CH)
    sidx2 = jnp.concatenate(
        [et * n + dst, jnp.full((pad,), n2, jnp.int32)]).reshape(nchunk, CH)

    sc = functools.partial(_sc_segment_sum, n_nodes=n,
                           nch_per_sub=nch_per_sub, acc_rows=acc_rows)

    sp1 = sc(feat.reshape(n2, 64), gsrc2, sidx2).reshape(2, 2, n, 64)
    h = _tc_layer(sp1, feat, W1, loopW1, b1.reshape(1, 128), relu=True)
    sp2 = sc(h.reshape(n2, 64), gsrc2, sidx2).reshape(2, 2, n, 64)
    out = _tc_layer(sp2, h, W2, loopW2, b2.reshape(1, 128), relu=False)
    return out


# speedup vs baseline: 1.2047x; 1.1553x over previous
import functools

import jax
import jax.numpy as jnp
from jax import lax
from jax.experimental import pallas as pl
from jax.experimental.pallas import tpu as pltpu
from jax.experimental.pallas import tpu_sc as plsc

NSUB = 16
NCORE = 2
CH = 128


def _sc_segment_sum(xcat, midx, *, n_nodes, nch_per_sub, acc_rows):
    n2 = 2 * n_nodes
    nchunk = nch_per_sub * NSUB
    zrep = acc_rows // (NSUB * CH)
    orows = n2 // NSUB
    mesh = plsc.VectorSubcoreMesh(core_axis_name="c", subcore_axis_name="s")

    @functools.partial(
        pl.kernel,
        mesh=mesh,
        out_type=jax.ShapeDtypeStruct((2 * n2, 64), jnp.float32),
        scratch_types=(
            [pltpu.VMEM((2, CH), jnp.int32)] +
            [pltpu.VMEM((CH, 64), jnp.float32)] +
            [pltpu.VMEM_SHARED((acc_rows, 64), jnp.float32)] +
            [pltpu.SemaphoreType.DMA]
        ),
        compiler_params=pltpu.CompilerParams(use_tc_tiling_on_sc=False),
    )
    def k(xcat_h, midx_h, out_h, gs, b0, acc, semg):
        c = lax.axis_index("c")
        s = lax.axis_index("s")
        T = nch_per_sub

        def zv(i, carry):
            b0[i // 4, pl.ds((i % 4) * 16, 16)] = jnp.zeros(
                (16,), jnp.float32)
            return carry
        lax.fori_loop(0, CH * 4, zv, 0)

        def za(i, carry):
            pltpu.sync_copy(b0, acc.at[pl.ds((s * zrep + i) * CH, CH)])
            return carry
        lax.fori_loop(0, zrep, za, 0)
        plsc.subcore_barrier()

        def step(j, carry):
            ch = c * nchunk + s * T + j
            pltpu.sync_copy(midx_h.at[pl.ds(2 * ch, 2)], gs)
            pltpu.async_copy(xcat_h.at[gs.at[0]], b0, semg).wait()
            pltpu.sync_copy(b0, acc.at[gs.at[1]], add=True)
            return carry
        lax.fori_loop(0, T, step, 0)
        plsc.subcore_barrier()

        pltpu.sync_copy(acc.at[pl.ds(s * orows, orows)],
                        out_h.at[pl.ds(c * n2 + s * orows, orows)])

    return k(xcat, midx)


def _tc_layer(sp, xin, W, lw, b, *, relu, blk=1000):
    n = xin.shape[0]
    grid = (n // blk,)

    def body(sp_ref, x_ref, w_ref, lw_ref, b_ref, o_ref):
        w = w_ref[...]
        acc = jnp.dot(x_ref[...], lw_ref[...],
                      preferred_element_type=jnp.float32)
        for c in range(2):
            for r in range(2):
                acc += jnp.dot(sp_ref[c, r], w[r, c * 64:(c + 1) * 64],
                               preferred_element_type=jnp.float32)
        acc += b_ref[...]
        if relu:
            acc = jnp.maximum(acc, 0.0)
        o_ref[...] = acc

    return pl.pallas_call(
        body,
        grid=grid,
        in_specs=[
            pl.BlockSpec((2, 2, blk, 64), lambda i: (0, 0, i, 0)),
            pl.BlockSpec((blk, 128), lambda i: (i, 0)),
            pl.BlockSpec((2, 128, 128), lambda i: (0, 0, 0)),
            pl.BlockSpec((128, 128), lambda i: (0, 0)),
            pl.BlockSpec((1, 128), lambda i: (0, 0)),
        ],
        out_specs=pl.BlockSpec((blk, 128), lambda i: (i, 0)),
        out_shape=jax.ShapeDtypeStruct((n, 128), jnp.float32),
    )(sp, xin, W, lw, b)


def kernel(feat, edge_index, etype, W1, loopW1, b1, W2, loopW2, b2):
    n = feat.shape[0]
    e = edge_index.shape[1]
    n2 = 2 * n

    nch_per_sub = -(-e // (NSUB * CH))
    e_pad = nch_per_sub * NSUB * CH
    nchunk = e_pad // CH
    acc_rows = -(-(n2 + 1) // (NSUB * CH)) * (NSUB * CH)

    src = edge_index[0].astype(jnp.int32)
    dst = edge_index[1].astype(jnp.int32)
    et = etype.astype(jnp.int32)
    pad = e_pad - e
    gidx = jnp.concatenate([2 * src, jnp.zeros((pad,), jnp.int32)])
    gsrc2 = jnp.concatenate([gidx, gidx + 1]).reshape(2 * nchunk, CH)
    sidx1 = jnp.concatenate(
        [et * n + dst, jnp.full((pad,), n2, jnp.int32)]).reshape(nchunk, CH)
    midx = jnp.stack(
        [gsrc2, jnp.concatenate([sidx1, sidx1])], axis=1
    ).reshape(4 * nchunk, CH)

    sc = functools.partial(_sc_segment_sum, n_nodes=n,
                           nch_per_sub=nch_per_sub, acc_rows=acc_rows)

    sp1 = sc(feat.reshape(n2, 64), midx).reshape(2, 2, n, 64)
    h = _tc_layer(sp1, feat, W1, loopW1, b1.reshape(1, 128), relu=True)
    sp2 = sc(h.reshape(n2, 64), midx).reshape(2, 2, n, 64)
    out = _tc_layer(sp2, h, W2, loopW2, b2.reshape(1, 128), relu=False)
    return out
